# baseline (speedup 1.0000x reference)
import functools

import jax
import jax.numpy as jnp
from jax import lax
from jax.experimental import pallas as pl
from jax.experimental.pallas import tpu as pltpu
from jax.experimental.pallas import tpu_sc as plsc

N = 10000
E = 320000
NPAD = 10240
STR = 640
NC = 2
NS = 16
NWK = NC * NS
EW = E // NWK
CH = 80
DW = CH * 128
RB = 1024

@functools.cache
def _mesh():
    return plsc.VectorSubcoreMesh(core_axis_name="c", subcore_axis_name="s",
                                  num_cores=NC, num_subcores=NS)


@functools.cache
def _make_deg():
    return functools.partial(
        pl.kernel,
        out_type=jax.ShapeDtypeStruct((NC, NPAD), jnp.float32),
        mesh=_mesh(),
        scratch_types=[
            pltpu.VMEM((CH, 128), jnp.int32),
            pltpu.VMEM((NPAD,), jnp.float32),
            pltpu.VMEM((STR,), jnp.float32),
            pltpu.VMEM((STR,), jnp.float32),
            pltpu.VMEM_SHARED((NS, NPAD), jnp.float32),
        ],
        compiler_params=pltpu.CompilerParams(needs_layout_passes=False,
                                             use_tc_tiling_on_sc=False),
    )(_deg_body)


def _deg_body(dstw, outd, ids, degl, acc, tmp, degsh):
    c = lax.axis_index("c")
    s = lax.axis_index("s")
    w = c * NS + s
    pltpu.sync_copy(dstw.at[w], ids)
    zeros = jnp.zeros((16,), jnp.float32)
    ones = jnp.ones((16,), jnp.float32)

    def _zero(i, _):
        degl[pl.ds(i * 16, 16)] = zeros
        return 0

    lax.fori_loop(0, NPAD // 16, _zero, 0)

    def _count(r, _):
        for k in range(8):
            idx = ids[r, pl.ds(k * 16, 16)]
            plsc.addupdate_scatter(degl, [idx], ones)
        return 0

    lax.fori_loop(0, CH, _count, 0)
    pltpu.sync_copy(degl, degsh.at[s])
    plsc.subcore_barrier()

    def _zacc(i, _):
        acc[pl.ds(i * 16, 16)] = zeros
        return 0

    lax.fori_loop(0, STR // 16, _zacc, 0)

    def _red(t, _):
        pltpu.sync_copy(degsh.at[t, pl.ds(s * STR, STR)], tmp)

        def _add(q, _):
            sl = pl.ds(q * 16, 16)
            acc[sl] = acc[sl] + tmp[sl]
            return 0

        lax.fori_loop(0, STR // 16, _add, 0)
        return 0

    lax.fori_loop(0, NS, _red, 0)
    pltpu.sync_copy(acc, outd.at[c, pl.ds(s * STR, STR)])


@functools.cache
def _make_scatter(D):
    @functools.partial(
        pl.kernel,
        out_type=jax.ShapeDtypeStruct((NC, NPAD, D), jnp.float32),
        mesh=_mesh(),
        scratch_types=[
            pltpu.VMEM((CH // 2, 128), jnp.int32),
            pltpu.VMEM((CH // 2, 128), jnp.int32),
            pltpu.VMEM((128, D), jnp.float32),
            pltpu.VMEM((128, D), jnp.float32),
            pltpu.VMEM((128, D), jnp.float32),
            pltpu.VMEM((128, D), jnp.float32),
            pltpu.VMEM_SHARED((NPAD, D), jnp.float32),
            pltpu.VMEM_SHARED((NPAD, D), jnp.float32),
            pltpu.SemaphoreType.DMA,
            pltpu.SemaphoreType.DMA,
            pltpu.SemaphoreType.DMA,
            pltpu.SemaphoreType.DMA,
            pltpu.SemaphoreType.DMA,
            pltpu.SemaphoreType.DMA,
            pltpu.SemaphoreType.DMA,
            pltpu.SemaphoreType.DMA,
        ],
        compiler_params=pltpu.CompilerParams(needs_layout_passes=False,
                                             use_tc_tiling_on_sc=False),
    )
    def _scatter(srcw, dstw, g, out, src_l, dst_l, bA0, bA1, bB0, bB1,
                 accum, gsh, gsA0, gsA1, ssA0, ssA1, gsB0, gsB1, ssB0, ssB1):
        c = lax.axis_index("c")
        s = lax.axis_index("s")
        w = c * NS + s
        CHH = CH // 2

        stripe = pl.ds(s * STR, STR)
        pltpu.async_copy(g.at[stripe], gsh.at[stripe], gsA1)

        zeros = jnp.zeros((16,), jnp.float32)

        def _zb(i, _):
            for k in range(D // 16):
                bA0[i, pl.ds(k * 16, 16)] = zeros
            return 0

        lax.fori_loop(0, 128, _zb, 0)
        for k in range(STR // 128):
            pltpu.sync_copy(bA0, accum.at[pl.ds(s * STR + k * 128, 128)])
        pltpu.make_async_copy(g.at[stripe], gsh.at[stripe], gsA1).wait()
        plsc.subcore_barrier()

        T = CHH // 4

        def _pair(t, off, b0, b1, gs0, gs1, ss0, ss1):
            j0 = 4 * t + off
            j1 = j0 + 2
            pltpu.make_async_copy(gsh.at[src_l.at[j0]], b0, gs0).wait()

            @pl.when(t > 0)
            def _():
                pltpu.make_async_copy(b1, accum.at[dst_l.at[j1]], ss1).wait()

            pltpu.async_copy(gsh.at[src_l.at[j1]], b1, gs1)
            pltpu.async_copy(b0, accum.at[dst_l.at[j0]], ss0, add=True)
            pltpu.make_async_copy(gsh.at[src_l.at[j1]], b1, gs1).wait()
            pltpu.make_async_copy(b0, accum.at[dst_l.at[j0]], ss0).wait()

            @pl.when(t < T - 1)
            def _():
                pltpu.async_copy(gsh.at[src_l.at[j0 + 4]], b0, gs0)

            pltpu.async_copy(b1, accum.at[dst_l.at[j1]], ss1, add=True)

        def _step(t, _):
            _pair(t, 0, bA0, bA1, gsA0, gsA1, ssA0, ssA1)
            _pair(t, 1, bB0, bB1, gsB0, gsB1, ssB0, ssB1)
            return 0

        for base in (0, CHH):
            pltpu.sync_copy(srcw.at[w, pl.ds(base, CHH)], src_l)
            pltpu.sync_copy(dstw.at[w, pl.ds(base, CHH)], dst_l)
            pltpu.async_copy(gsh.at[src_l.at[0]], bA0, gsA0)
            pltpu.async_copy(gsh.at[src_l.at[1]], bB0, gsB0)
            lax.fori_loop(0, T, _step, 0)
            pltpu.make_async_copy(bA1, accum.at[dst_l.at[CHH - 2]],
                                  ssA1).wait()
            pltpu.make_async_copy(bB1, accum.at[dst_l.at[CHH - 1]],
                                  ssB1).wait()
        plsc.subcore_barrier()
        pltpu.sync_copy(
            accum.at[pl.ds(s * STR, STR)], out.at[c, pl.ds(s * STR, STR)]
        )

    return _scatter


def _dinv_of(degp_blk):
    deg = degp_blk[0, :] + degp_blk[1, :] + 1.0
    return lax.rsqrt(jnp.maximum(deg, 1.0))


def _enc1_body(x_ref, w1_ref, degp_ref, out_ref):
    dinv = _dinv_of(degp_ref)
    h = jnp.dot(x_ref[...], w1_ref[...], preferred_element_type=jnp.float32,
                precision=lax.Precision.HIGHEST)
    out_ref[...] = h * dinv[:, None]


def _enc1(x, W1, degp):
    return pl.pallas_call(
        _enc1_body,
        grid=(NPAD // RB,),
        in_specs=[
            pl.BlockSpec((RB, 128), lambda i: (i, 0)),
            pl.BlockSpec((128, 64), lambda i: (0, 0)),
            pl.BlockSpec((NC, RB), lambda i: (0, i)),
        ],
        out_specs=pl.BlockSpec((RB, 64), lambda i: (i, 0)),
        out_shape=jax.ShapeDtypeStruct((NPAD, 64), jnp.float32),
    )(x, W1, degp)


def _enc2_body(s1_ref, g1_ref, degp_ref, b1_ref, w2_ref, out_ref):
    dinv = _dinv_of(degp_ref)
    conv = dinv[:, None] * (s1_ref[0] + s1_ref[1] + g1_ref[...]) + b1_ref[...]
    h = jnp.maximum(conv, 0.0)
    t = jnp.dot(h, w2_ref[...], preferred_element_type=jnp.float32,
                precision=lax.Precision.HIGHEST)
    out_ref[...] = t * dinv[:, None]


def _enc2(S1, g1, degp, b1, W2):
    return pl.pallas_call(
        _enc2_body,
        grid=(NPAD // RB,),
        in_specs=[
            pl.BlockSpec((NC, RB, 64), lambda i: (0, i, 0)),
            pl.BlockSpec((RB, 64), lambda i: (i, 0)),
            pl.BlockSpec((NC, RB), lambda i: (0, i)),
            pl.BlockSpec((1, 64), lambda i: (0, 0)),
            pl.BlockSpec((64, 32), lambda i: (0, 0)),
        ],
        out_specs=pl.BlockSpec((RB, 32), lambda i: (i, 0)),
        out_shape=jax.ShapeDtypeStruct((NPAD, 32), jnp.float32),
    )(S1, g1, degp, b1, W2)


def _dec_body(s2_ref, g2_ref, degp_ref, b2_ref, wd1_ref, bd1_ref, wd2_ref,
              bd2_ref, eps_ref, dec_ref, mu_ref, lv_ref):
    dinv = _dinv_of(degp_ref)
    enc = dinv[:, None] * (s2_ref[0] + s2_ref[1] + g2_ref[...]) + b2_ref[...]
    mu = enc[:, :16]
    lv = enc[:, 16:]
    mu_ref[...] = mu
    lv_ref[...] = lv
    std = jnp.exp(0.5 * lv)
    z = mu + eps_ref[...] * std
    d = jnp.dot(z, wd1_ref[...], preferred_element_type=jnp.float32,
                precision=lax.Precision.HIGHEST) + bd1_ref[...]
    d = jnp.maximum(d, 0.0)
    o = jnp.dot(d, wd2_ref[...], preferred_element_type=jnp.float32,
                precision=lax.Precision.HIGHEST) + bd2_ref[...]
    dec_ref[...] = jax.nn.sigmoid(o)


def _dec(S2, g2, degp, b2, Wd1, bd1, Wd2, bd2, eps_p):
    return pl.pallas_call(
        _dec_body,
        grid=(NPAD // RB,),
        in_specs=[
            pl.BlockSpec((NC, RB, 32), lambda i: (0, i, 0)),
            pl.BlockSpec((RB, 32), lambda i: (i, 0)),
            pl.BlockSpec((NC, RB), lambda i: (0, i)),
            pl.BlockSpec((1, 32), lambda i: (0, 0)),
            pl.BlockSpec((16, 64), lambda i: (0, 0)),
            pl.BlockSpec((1, 64), lambda i: (0, 0)),
            pl.BlockSpec((64, 128), lambda i: (0, 0)),
            pl.BlockSpec((1, 128), lambda i: (0, 0)),
            pl.BlockSpec((RB, 16), lambda i: (i, 0)),
        ],
        out_specs=[
            pl.BlockSpec((RB, 128), lambda i: (i, 0)),
            pl.BlockSpec((RB, 16), lambda i: (i, 0)),
            pl.BlockSpec((RB, 16), lambda i: (i, 0)),
        ],
        out_shape=[
            jax.ShapeDtypeStruct((N, 128), jnp.float32),
            jax.ShapeDtypeStruct((N, 16), jnp.float32),
            jax.ShapeDtypeStruct((N, 16), jnp.float32),
        ],
    )(S2, g2, degp, b2, Wd1, bd1, Wd2, bd2, eps_p)


@jax.jit
def kernel(x, edge_index, W1, b1, W2, b2, Wd1, bd1, Wd2, bd2):
    ei = edge_index.astype(jnp.int32)
    srcw = jnp.pad(ei[0].reshape(NWK, EW), ((0, 0), (0, DW - EW)))
    dstw = jnp.pad(ei[1].reshape(NWK, EW), ((0, 0), (0, DW - EW)),
                   constant_values=N)
    srcw = srcw.reshape(NWK, CH, 128)
    dstw = dstw.reshape(NWK, CH, 128)

    eps = jax.random.normal(jax.random.key(42), (N, 16), dtype=jnp.float32)

    degp = _make_deg()(dstw)
    g1 = _enc1(x, W1, degp)
    S1 = _make_scatter(64)(srcw, dstw, g1)
    g2 = _enc2(S1, g1, degp, b1.reshape(1, 64), W2)
    S2 = _make_scatter(32)(srcw, dstw, g2)
    dec, mu, lv = _dec(S2, g2, degp, b2.reshape(1, 32), Wd1,
                       bd1.reshape(1, 64), Wd2, bd2.reshape(1, 128), eps)
    return (dec, mu, lv)


# speedup vs baseline: 1.0257x; 1.0257x over previous
import functools

import jax
import jax.numpy as jnp
import numpy as np
from jax import lax
from jax.experimental import pallas as pl
from jax.experimental.pallas import tpu as pltpu
from jax.experimental.pallas import tpu_sc as plsc

N = 10000
E = 320000
NPAD = 10240
STR = 640
NC = 2
NS = 16
NWK = NC * NS
EW = E // NWK
CH = 80
DW = CH * 128
RB = 2048


_EPS = np.asarray(
    jax.random.normal(jax.random.key(42), (N, 16), dtype=jnp.float32))

@functools.cache
def _mesh():
    return plsc.VectorSubcoreMesh(core_axis_name="c", subcore_axis_name="s",
                                  num_cores=NC, num_subcores=NS)


@functools.cache
def _make_deg():
    return functools.partial(
        pl.kernel,
        out_type=jax.ShapeDtypeStruct((NC, NPAD), jnp.float32),
        mesh=_mesh(),
        scratch_types=[
            pltpu.VMEM((CH, 128), jnp.int32),
            pltpu.VMEM((NPAD,), jnp.float32),
            pltpu.VMEM((STR,), jnp.float32),
            pltpu.VMEM((STR,), jnp.float32),
            pltpu.VMEM_SHARED((NS, NPAD), jnp.float32),
        ],
        compiler_params=pltpu.CompilerParams(needs_layout_passes=False,
                                             use_tc_tiling_on_sc=False),
    )(_deg_body)


def _deg_body(dstw, outd, ids, degl, acc, tmp, degsh):
    c = lax.axis_index("c")
    s = lax.axis_index("s")
    w = c * NS + s
    pltpu.sync_copy(dstw.at[w], ids)
    zeros = jnp.zeros((16,), jnp.float32)
    ones = jnp.ones((16,), jnp.float32)

    def _zero(i, _):
        degl[pl.ds(i * 16, 16)] = zeros
        return 0

    lax.fori_loop(0, NPAD // 16, _zero, 0)

    def _count(r, _):
        for k in range(8):
            idx = ids[r, pl.ds(k * 16, 16)]
            plsc.addupdate_scatter(degl, [idx], ones)
        return 0

    lax.fori_loop(0, CH, _count, 0)
    pltpu.sync_copy(degl, degsh.at[s])
    plsc.subcore_barrier()

    def _zacc(i, _):
        acc[pl.ds(i * 16, 16)] = zeros
        return 0

    lax.fori_loop(0, STR // 16, _zacc, 0)

    def _red(t, _):
        pltpu.sync_copy(degsh.at[t, pl.ds(s * STR, STR)], tmp)

        def _add(q, _):
            sl = pl.ds(q * 16, 16)
            acc[sl] = acc[sl] + tmp[sl]
            return 0

        lax.fori_loop(0, STR // 16, _add, 0)
        return 0

    lax.fori_loop(0, NS, _red, 0)
    pltpu.sync_copy(acc, outd.at[c, pl.ds(s * STR, STR)])


@functools.cache
def _make_scatter(D):
    @functools.partial(
        pl.kernel,
        out_type=jax.ShapeDtypeStruct((NC, NPAD, D), jnp.float32),
        mesh=_mesh(),
        scratch_types=[
            pltpu.VMEM((CH // 2, 128), jnp.int32),
            pltpu.VMEM((CH // 2, 128), jnp.int32),
            pltpu.VMEM((128, D), jnp.float32),
            pltpu.VMEM((128, D), jnp.float32),
            pltpu.VMEM((128, D), jnp.float32),
            pltpu.VMEM((128, D), jnp.float32),
            pltpu.VMEM_SHARED((NPAD, D), jnp.float32),
            pltpu.VMEM_SHARED((NPAD, D), jnp.float32),
            pltpu.SemaphoreType.DMA,
            pltpu.SemaphoreType.DMA,
            pltpu.SemaphoreType.DMA,
            pltpu.SemaphoreType.DMA,
            pltpu.SemaphoreType.DMA,
            pltpu.SemaphoreType.DMA,
            pltpu.SemaphoreType.DMA,
            pltpu.SemaphoreType.DMA,
        ],
        compiler_params=pltpu.CompilerParams(needs_layout_passes=False,
                                             use_tc_tiling_on_sc=False),
    )
    def _scatter(srcw, dstw, g, out, src_l, dst_l, bA0, bA1, bB0, bB1,
                 accum, gsh, gsA0, gsA1, ssA0, ssA1, gsB0, gsB1, ssB0, ssB1):
        c = lax.axis_index("c")
        s = lax.axis_index("s")
        w = c * NS + s
        CHH = CH // 2

        stripe = pl.ds(s * STR, STR)
        pltpu.async_copy(g.at[stripe], gsh.at[stripe], gsA1)

        zeros = jnp.zeros((16,), jnp.float32)

        def _zb(i, _):
            for k in range(D // 16):
                bA0[i, pl.ds(k * 16, 16)] = zeros
            return 0

        lax.fori_loop(0, 128, _zb, 0)
        for k in range(STR // 128):
            pltpu.sync_copy(bA0, accum.at[pl.ds(s * STR + k * 128, 128)])
        pltpu.make_async_copy(g.at[stripe], gsh.at[stripe], gsA1).wait()
        plsc.subcore_barrier()

        T = CHH // 4

        def _pair(t, off, b0, b1, gs0, gs1, ss0, ss1):
            j0 = 4 * t + off
            j1 = j0 + 2
            pltpu.make_async_copy(gsh.at[src_l.at[j0]], b0, gs0).wait()

            @pl.when(t > 0)
            def _():
                pltpu.make_async_copy(b1, accum.at[dst_l.at[j1]], ss1).wait()

            pltpu.async_copy(gsh.at[src_l.at[j1]], b1, gs1)
            pltpu.async_copy(b0, accum.at[dst_l.at[j0]], ss0, add=True)
            pltpu.make_async_copy(gsh.at[src_l.at[j1]], b1, gs1).wait()
            pltpu.make_async_copy(b0, accum.at[dst_l.at[j0]], ss0).wait()

            @pl.when(t < T - 1)
            def _():
                pltpu.async_copy(gsh.at[src_l.at[j0 + 4]], b0, gs0)

            pltpu.async_copy(b1, accum.at[dst_l.at[j1]], ss1, add=True)

        def _step(t, _):
            _pair(t, 0, bA0, bA1, gsA0, gsA1, ssA0, ssA1)
            _pair(t, 1, bB0, bB1, gsB0, gsB1, ssB0, ssB1)
            return 0

        for base in (0, CHH):
            pltpu.sync_copy(srcw.at[w, pl.ds(base, CHH)], src_l)
            pltpu.sync_copy(dstw.at[w, pl.ds(base, CHH)], dst_l)
            pltpu.async_copy(gsh.at[src_l.at[0]], bA0, gsA0)
            pltpu.async_copy(gsh.at[src_l.at[1]], bB0, gsB0)
            lax.fori_loop(0, T, _step, 0)
            pltpu.make_async_copy(bA1, accum.at[dst_l.at[CHH - 2]],
                                  ssA1).wait()
            pltpu.make_async_copy(bB1, accum.at[dst_l.at[CHH - 1]],
                                  ssB1).wait()
        plsc.subcore_barrier()
        pltpu.sync_copy(
            accum.at[pl.ds(s * STR, STR)], out.at[c, pl.ds(s * STR, STR)]
        )

    return _scatter


def _dinv_of(degp_blk):
    deg = degp_blk[0, :] + degp_blk[1, :] + 1.0
    return lax.rsqrt(jnp.maximum(deg, 1.0))


def _enc1_body(x_ref, w1_ref, degp_ref, out_ref):
    dinv = _dinv_of(degp_ref)
    h = jnp.dot(x_ref[...], w1_ref[...], preferred_element_type=jnp.float32,
                precision=lax.Precision.HIGHEST)
    out_ref[...] = h * dinv[:, None]


def _enc1(x, W1, degp):
    return pl.pallas_call(
        _enc1_body,
        grid=(NPAD // RB,),
        in_specs=[
            pl.BlockSpec((RB, 128), lambda i: (i, 0)),
            pl.BlockSpec((128, 64), lambda i: (0, 0)),
            pl.BlockSpec((NC, RB), lambda i: (0, i)),
        ],
        out_specs=pl.BlockSpec((RB, 64), lambda i: (i, 0)),
        out_shape=jax.ShapeDtypeStruct((NPAD, 64), jnp.float32),
    )(x, W1, degp)


def _enc2_body(s1_ref, g1_ref, degp_ref, b1_ref, w2_ref, out_ref):
    dinv = _dinv_of(degp_ref)
    conv = dinv[:, None] * (s1_ref[0] + s1_ref[1] + g1_ref[...]) + b1_ref[...]
    h = jnp.maximum(conv, 0.0)
    t = jnp.dot(h, w2_ref[...], preferred_element_type=jnp.float32,
                precision=lax.Precision.HIGHEST)
    out_ref[...] = t * dinv[:, None]


def _enc2(S1, g1, degp, b1, W2):
    return pl.pallas_call(
        _enc2_body,
        grid=(NPAD // RB,),
        in_specs=[
            pl.BlockSpec((NC, RB, 64), lambda i: (0, i, 0)),
            pl.BlockSpec((RB, 64), lambda i: (i, 0)),
            pl.BlockSpec((NC, RB), lambda i: (0, i)),
            pl.BlockSpec((1, 64), lambda i: (0, 0)),
            pl.BlockSpec((64, 32), lambda i: (0, 0)),
        ],
        out_specs=pl.BlockSpec((RB, 32), lambda i: (i, 0)),
        out_shape=jax.ShapeDtypeStruct((NPAD, 32), jnp.float32),
    )(S1, g1, degp, b1, W2)


def _dec_body(s2_ref, g2_ref, degp_ref, b2_ref, wd1_ref, bd1_ref, wd2_ref,
              bd2_ref, eps_ref, dec_ref, mu_ref, lv_ref):
    dinv = _dinv_of(degp_ref)
    enc = dinv[:, None] * (s2_ref[0] + s2_ref[1] + g2_ref[...]) + b2_ref[...]
    mu = enc[:, :16]
    lv = enc[:, 16:]
    mu_ref[...] = mu
    lv_ref[...] = lv
    std = jnp.exp(0.5 * lv)
    z = mu + eps_ref[...] * std
    d = jnp.dot(z, wd1_ref[...], preferred_element_type=jnp.float32,
                precision=lax.Precision.HIGHEST) + bd1_ref[...]
    d = jnp.maximum(d, 0.0)
    o = jnp.dot(d, wd2_ref[...], preferred_element_type=jnp.float32,
                precision=lax.Precision.HIGHEST) + bd2_ref[...]
    dec_ref[...] = jax.nn.sigmoid(o)


def _dec(S2, g2, degp, b2, Wd1, bd1, Wd2, bd2, eps_p):
    return pl.pallas_call(
        _dec_body,
        grid=(NPAD // RB,),
        in_specs=[
            pl.BlockSpec((NC, RB, 32), lambda i: (0, i, 0)),
            pl.BlockSpec((RB, 32), lambda i: (i, 0)),
            pl.BlockSpec((NC, RB), lambda i: (0, i)),
            pl.BlockSpec((1, 32), lambda i: (0, 0)),
            pl.BlockSpec((16, 64), lambda i: (0, 0)),
            pl.BlockSpec((1, 64), lambda i: (0, 0)),
            pl.BlockSpec((64, 128), lambda i: (0, 0)),
            pl.BlockSpec((1, 128), lambda i: (0, 0)),
            pl.BlockSpec((RB, 16), lambda i: (i, 0)),
        ],
        out_specs=[
            pl.BlockSpec((RB, 128), lambda i: (i, 0)),
            pl.BlockSpec((RB, 16), lambda i: (i, 0)),
            pl.BlockSpec((RB, 16), lambda i: (i, 0)),
        ],
        out_shape=[
            jax.ShapeDtypeStruct((N, 128), jnp.float32),
            jax.ShapeDtypeStruct((N, 16), jnp.float32),
            jax.ShapeDtypeStruct((N, 16), jnp.float32),
        ],
    )(S2, g2, degp, b2, Wd1, bd1, Wd2, bd2, eps_p)


@jax.jit
def kernel(x, edge_index, W1, b1, W2, b2, Wd1, bd1, Wd2, bd2):
    ei = edge_index.astype(jnp.int32)
    srcw = jnp.pad(ei[0].reshape(NWK, EW), ((0, 0), (0, DW - EW)))
    dstw = jnp.pad(ei[1].reshape(NWK, EW), ((0, 0), (0, DW - EW)),
                   constant_values=N)
    srcw = srcw.reshape(NWK, CH, 128)
    dstw = dstw.reshape(NWK, CH, 128)

    eps = jnp.asarray(_EPS)

    degp = _make_deg()(dstw)
    g1 = _enc1(x, W1, degp)
    S1 = _make_scatter(64)(srcw, dstw, g1)
    g2 = _enc2(S1, g1, degp, b1.reshape(1, 64), W2)
    S2 = _make_scatter(32)(srcw, dstw, g2)
    dec, mu, lv = _dec(S2, g2, degp, b2.reshape(1, 32), Wd1,
                       bd1.reshape(1, 64), Wd2, bd2.reshape(1, 128), eps)
    return (dec, mu, lv)


# speedup vs baseline: 1.0517x; 1.0253x over previous
import functools

import jax
import jax.numpy as jnp
import numpy as np
from jax import lax
from jax.experimental import pallas as pl
from jax.experimental.pallas import tpu as pltpu
from jax.experimental.pallas import tpu_sc as plsc

N = 10000
E = 320000
NPAD = 10240
STR = 640
NC = 2
NS = 16
NWK = NC * NS
NCH = E // 128
CHB = NCH // NWK
XW = NCH - CHB * NWK
CHM = CHB + 1
RB = 2048

_EPS = np.asarray(
    jax.random.normal(jax.random.key(42), (N, 16), dtype=jnp.float32))


@functools.cache
def _mesh():
    return plsc.VectorSubcoreMesh(core_axis_name="c", subcore_axis_name="s",
                                  num_cores=NC, num_subcores=NS)


def _worker_chunks(w):
    cbase = CHB * w + jnp.minimum(w, XW)
    extra = (w < XW).astype(jnp.int32)
    return cbase, extra


def _load_idx(hbm, vmem, cbase, w):
    @pl.when(w < XW)
    def _():
        pltpu.sync_copy(hbm.at[pl.ds(cbase, CHM)], vmem)

    @pl.when(w >= XW)
    def _():
        pltpu.sync_copy(hbm.at[pl.ds(cbase, CHB)], vmem.at[pl.ds(0, CHB)])


@functools.cache
def _make_deg():
    return functools.partial(
        pl.kernel,
        out_type=jax.ShapeDtypeStruct((NC, NPAD), jnp.float32),
        mesh=_mesh(),
        scratch_types=[
            pltpu.VMEM((CHM, 128), jnp.int32),
            pltpu.VMEM((NPAD,), jnp.float32),
            pltpu.VMEM((STR,), jnp.float32),
            pltpu.VMEM((STR,), jnp.float32),
            pltpu.VMEM_SHARED((NS, NPAD), jnp.float32),
        ],
        compiler_params=pltpu.CompilerParams(needs_layout_passes=False,
                                             use_tc_tiling_on_sc=False),
    )(_deg_body)


def _deg_body(dstw, outd, ids, degl, acc, tmp, degsh):
    c = lax.axis_index("c")
    s = lax.axis_index("s")
    w = c * NS + s
    cbase, extra = _worker_chunks(w)
    _load_idx(dstw, ids, cbase, w)
    zeros = jnp.zeros((16,), jnp.float32)
    ones = jnp.ones((16,), jnp.float32)

    def _zero(i, _):
        degl[pl.ds(i * 16, 16)] = zeros
        return 0

    lax.fori_loop(0, NPAD // 16, _zero, 0)

    def _count(r, _):
        for k in range(8):
            idx = ids[r, pl.ds(k * 16, 16)]
            plsc.addupdate_scatter(degl, [idx], ones)
        return 0

    lax.fori_loop(0, CHB + extra, _count, 0)
    pltpu.sync_copy(degl, degsh.at[s])
    plsc.subcore_barrier()

    def _zacc(i, _):
        acc[pl.ds(i * 16, 16)] = zeros
        return 0

    lax.fori_loop(0, STR // 16, _zacc, 0)

    def _red(t, _):
        pltpu.sync_copy(degsh.at[t, pl.ds(s * STR, STR)], tmp)

        def _add(q, _):
            sl = pl.ds(q * 16, 16)
            acc[sl] = acc[sl] + tmp[sl]
            return 0

        lax.fori_loop(0, STR // 16, _add, 0)
        return 0

    lax.fori_loop(0, NS, _red, 0)
    pltpu.sync_copy(acc, outd.at[c, pl.ds(s * STR, STR)])


@functools.cache
def _make_scatter(D):
    @functools.partial(
        pl.kernel,
        out_type=jax.ShapeDtypeStruct((NC, NPAD, D), jnp.float32),
        mesh=_mesh(),
        scratch_types=[
            pltpu.VMEM((CHM, 128), jnp.int32),
            pltpu.VMEM((CHM, 128), jnp.int32),
            pltpu.VMEM((128, D), jnp.float32),
            pltpu.VMEM((128, D), jnp.float32),
            pltpu.VMEM_SHARED((NPAD, D), jnp.float32),
            pltpu.VMEM_SHARED((NPAD, D), jnp.float32),
            pltpu.SemaphoreType.DMA,
            pltpu.SemaphoreType.DMA,
            pltpu.SemaphoreType.DMA,
            pltpu.SemaphoreType.DMA,
        ],
        compiler_params=pltpu.CompilerParams(needs_layout_passes=False,
                                             use_tc_tiling_on_sc=False),
    )
    def _scatter(srcw, dstw, g, out, src_l, dst_l, buf0, buf1, accum, gsh,
                 gsem0, gsem1, ssem0, ssem1):
        c = lax.axis_index("c")
        s = lax.axis_index("s")
        w = c * NS + s
        cbase, extra = _worker_chunks(w)
        _load_idx(srcw, src_l, cbase, w)
        _load_idx(dstw, dst_l, cbase, w)

        stripe = pl.ds(s * STR, STR)
        pltpu.async_copy(g.at[stripe], gsh.at[stripe], gsem1)

        zeros = jnp.zeros((16,), jnp.float32)

        def _zb(i, _):
            for k in range(D // 16):
                buf0[i, pl.ds(k * 16, 16)] = zeros
            return 0

        lax.fori_loop(0, 128, _zb, 0)
        for k in range(STR // 128):
            pltpu.sync_copy(buf0, accum.at[pl.ds(s * STR + k * 128, 128)])
        pltpu.make_async_copy(g.at[stripe], gsh.at[stripe], gsem1).wait()
        plsc.subcore_barrier()

        pltpu.async_copy(gsh.at[src_l.at[0]], buf0, gsem0)
        T = CHB // 2

        def _edge_pair(t, _):
            j0 = 2 * t
            j1 = 2 * t + 1
            pltpu.make_async_copy(gsh.at[src_l.at[j0]], buf0, gsem0).wait()

            @pl.when(t > 0)
            def _():
                pltpu.make_async_copy(buf1, accum.at[dst_l.at[j1]],
                                      ssem1).wait()

            pltpu.async_copy(gsh.at[src_l.at[j1]], buf1, gsem1)
            pltpu.async_copy(buf0, accum.at[dst_l.at[j0]], ssem0, add=True)
            pltpu.make_async_copy(gsh.at[src_l.at[j1]], buf1, gsem1).wait()
            pltpu.make_async_copy(buf0, accum.at[dst_l.at[j0]], ssem0).wait()

            @pl.when(t < T - 1)
            def _():
                pltpu.async_copy(gsh.at[src_l.at[j0 + 2]], buf0, gsem0)

            pltpu.async_copy(buf1, accum.at[dst_l.at[j1]], ssem1, add=True)
            return 0

        lax.fori_loop(0, T, _edge_pair, 0)
        pltpu.make_async_copy(buf1, accum.at[dst_l.at[CHB - 1]], ssem1).wait()

        @pl.when(extra > 0)
        def _():
            pltpu.async_copy(gsh.at[src_l.at[CHB]], buf0, gsem0).wait()
            pltpu.sync_copy(buf0, accum.at[dst_l.at[CHB]], add=True)

        plsc.subcore_barrier()
        pltpu.sync_copy(
            accum.at[pl.ds(s * STR, STR)], out.at[c, pl.ds(s * STR, STR)]
        )

    return _scatter


def _dinv_of(degp_blk):
    deg = degp_blk[0, :] + degp_blk[1, :] + 1.0
    return lax.rsqrt(jnp.maximum(deg, 1.0))


def _enc1_body(x_ref, w1_ref, degp_ref, out_ref):
    dinv = _dinv_of(degp_ref)
    h = jnp.dot(x_ref[...], w1_ref[...], preferred_element_type=jnp.float32,
                precision=lax.Precision.HIGHEST)
    out_ref[...] = h * dinv[:, None]


def _enc1(x, W1, degp):
    return pl.pallas_call(
        _enc1_body,
        grid=(NPAD // RB,),
        in_specs=[
            pl.BlockSpec((RB, 128), lambda i: (i, 0)),
            pl.BlockSpec((128, 64), lambda i: (0, 0)),
            pl.BlockSpec((NC, RB), lambda i: (0, i)),
        ],
        out_specs=pl.BlockSpec((RB, 64), lambda i: (i, 0)),
        out_shape=jax.ShapeDtypeStruct((NPAD, 64), jnp.float32),
    )(x, W1, degp)


def _enc2_body(s1_ref, g1_ref, degp_ref, b1_ref, w2_ref, out_ref):
    dinv = _dinv_of(degp_ref)
    conv = dinv[:, None] * (s1_ref[0] + s1_ref[1] + g1_ref[...]) + b1_ref[...]
    h = jnp.maximum(conv, 0.0)
    t = jnp.dot(h, w2_ref[...], preferred_element_type=jnp.float32,
                precision=lax.Precision.HIGHEST)
    out_ref[...] = t * dinv[:, None]


def _enc2(S1, g1, degp, b1, W2):
    return pl.pallas_call(
        _enc2_body,
        grid=(NPAD // RB,),
        in_specs=[
            pl.BlockSpec((NC, RB, 64), lambda i: (0, i, 0)),
            pl.BlockSpec((RB, 64), lambda i: (i, 0)),
            pl.BlockSpec((NC, RB), lambda i: (0, i)),
            pl.BlockSpec((1, 64), lambda i: (0, 0)),
            pl.BlockSpec((64, 32), lambda i: (0, 0)),
        ],
        out_specs=pl.BlockSpec((RB, 32), lambda i: (i, 0)),
        out_shape=jax.ShapeDtypeStruct((NPAD, 32), jnp.float32),
    )(S1, g1, degp, b1, W2)


def _dec_body(s2_ref, g2_ref, degp_ref, b2_ref, wd1_ref, bd1_ref, wd2_ref,
              bd2_ref, eps_ref, dec_ref, mu_ref, lv_ref):
    dinv = _dinv_of(degp_ref)
    enc = dinv[:, None] * (s2_ref[0] + s2_ref[1] + g2_ref[...]) + b2_ref[...]
    mu = enc[:, :16]
    lv = enc[:, 16:]
    mu_ref[...] = mu
    lv_ref[...] = lv
    std = jnp.exp(0.5 * lv)
    z = mu + eps_ref[...] * std
    d = jnp.dot(z, wd1_ref[...], preferred_element_type=jnp.float32,
                precision=lax.Precision.HIGHEST) + bd1_ref[...]
    d = jnp.maximum(d, 0.0)
    o = jnp.dot(d, wd2_ref[...], preferred_element_type=jnp.float32,
                precision=lax.Precision.HIGHEST) + bd2_ref[...]
    dec_ref[...] = jax.nn.sigmoid(o)


def _dec(S2, g2, degp, b2, Wd1, bd1, Wd2, bd2, eps):
    return pl.pallas_call(
        _dec_body,
        grid=(NPAD // RB,),
        in_specs=[
            pl.BlockSpec((NC, RB, 32), lambda i: (0, i, 0)),
            pl.BlockSpec((RB, 32), lambda i: (i, 0)),
            pl.BlockSpec((NC, RB), lambda i: (0, i)),
            pl.BlockSpec((1, 32), lambda i: (0, 0)),
            pl.BlockSpec((16, 64), lambda i: (0, 0)),
            pl.BlockSpec((1, 64), lambda i: (0, 0)),
            pl.BlockSpec((64, 128), lambda i: (0, 0)),
            pl.BlockSpec((1, 128), lambda i: (0, 0)),
            pl.BlockSpec((RB, 16), lambda i: (i, 0)),
        ],
        out_specs=[
            pl.BlockSpec((RB, 128), lambda i: (i, 0)),
            pl.BlockSpec((RB, 16), lambda i: (i, 0)),
            pl.BlockSpec((RB, 16), lambda i: (i, 0)),
        ],
        out_shape=[
            jax.ShapeDtypeStruct((N, 128), jnp.float32),
            jax.ShapeDtypeStruct((N, 16), jnp.float32),
            jax.ShapeDtypeStruct((N, 16), jnp.float32),
        ],
    )(S2, g2, degp, b2, Wd1, bd1, Wd2, bd2, eps)


@jax.jit
def kernel(x, edge_index, W1, b1, W2, b2, Wd1, bd1, Wd2, bd2):
    ei = edge_index.astype(jnp.int32)
    srcw = ei[0].reshape(NCH, 128)
    dstw = ei[1].reshape(NCH, 128)
    eps = jnp.asarray(_EPS)

    degp = _make_deg()(dstw)
    g1 = _enc1(x, W1, degp)
    S1 = _make_scatter(64)(srcw, dstw, g1)
    g2 = _enc2(S1, g1, degp, b1.reshape(1, 64), W2)
    S2 = _make_scatter(32)(srcw, dstw, g2)
    dec, mu, lv = _dec(S2, g2, degp, b2.reshape(1, 32), Wd1,
                       bd1.reshape(1, 64), Wd2, bd2.reshape(1, 128), eps)
    return (dec, mu, lv)


# speedup vs baseline: 1.0524x; 1.0007x over previous
import functools

import jax
import jax.numpy as jnp
import numpy as np
from jax import lax
from jax.experimental import pallas as pl
from jax.experimental.pallas import tpu as pltpu
from jax.experimental.pallas import tpu_sc as plsc

N = 10000
E = 320000
NPAD = 10240
STR = 640
NC = 2
NS = 16
NWK = NC * NS
NCH = E // 128
CHB = NCH // NWK
XW = NCH - CHB * NWK
CHM = CHB + 1
RB = 2048

_EPS = np.asarray(
    jax.random.normal(jax.random.key(42), (N, 16), dtype=jnp.float32))


@functools.cache
def _mesh():
    return plsc.VectorSubcoreMesh(core_axis_name="c", subcore_axis_name="s",
                                  num_cores=NC, num_subcores=NS)


def _worker_chunks(w):
    cbase = CHB * w + jnp.minimum(w, XW)
    extra = (w < XW).astype(jnp.int32)
    return cbase, extra


def _load_idx(hbm, vmem, cbase, w):
    @pl.when(w < XW)
    def _():
        pltpu.sync_copy(hbm.at[pl.ds(cbase, CHM)], vmem)

    @pl.when(w >= XW)
    def _():
        pltpu.sync_copy(hbm.at[pl.ds(cbase, CHB)], vmem.at[pl.ds(0, CHB)])


@functools.cache
def _make_deg():
    return functools.partial(
        pl.kernel,
        out_type=jax.ShapeDtypeStruct((NC, NPAD), jnp.float32),
        mesh=_mesh(),
        scratch_types=[
            pltpu.VMEM((CHM, 128), jnp.int32),
            pltpu.VMEM((NPAD,), jnp.float32),
            pltpu.VMEM((STR,), jnp.float32),
            pltpu.VMEM((STR,), jnp.float32),
            pltpu.VMEM_SHARED((NS, NPAD), jnp.float32),
        ],
        compiler_params=pltpu.CompilerParams(needs_layout_passes=False,
                                             use_tc_tiling_on_sc=False),
    )(_deg_body)


def _deg_body(dstw, outd, ids, degl, acc, tmp, degsh):
    c = lax.axis_index("c")
    s = lax.axis_index("s")
    w = c * NS + s
    cbase, extra = _worker_chunks(w)
    _load_idx(dstw, ids, cbase, w)
    zeros = jnp.zeros((16,), jnp.float32)
    ones = jnp.ones((16,), jnp.float32)

    def _zero(i, _):
        degl[pl.ds(i * 16, 16)] = zeros
        return 0

    lax.fori_loop(0, NPAD // 16, _zero, 0)

    def _count(r, _):
        for k in range(8):
            idx = ids[r, pl.ds(k * 16, 16)]
            plsc.addupdate_scatter(degl, [idx], ones)
        return 0

    lax.fori_loop(0, CHB + extra, _count, 0)
    pltpu.sync_copy(degl, degsh.at[s])
    plsc.subcore_barrier()

    def _zacc(i, _):
        acc[pl.ds(i * 16, 16)] = zeros
        return 0

    lax.fori_loop(0, STR // 16, _zacc, 0)

    def _red(t, _):
        pltpu.sync_copy(degsh.at[t, pl.ds(s * STR, STR)], tmp)

        def _add(q, _):
            sl = pl.ds(q * 16, 16)
            acc[sl] = acc[sl] + tmp[sl]
            return 0

        lax.fori_loop(0, STR // 16, _add, 0)
        return 0

    lax.fori_loop(0, NS, _red, 0)
    pltpu.sync_copy(acc, outd.at[c, pl.ds(s * STR, STR)])


@functools.cache
def _make_scatter(D):
    @functools.partial(
        pl.kernel,
        out_type=jax.ShapeDtypeStruct((NC, NPAD, D), jnp.float32),
        mesh=_mesh(),
        scratch_types=[
            pltpu.VMEM((CHM, 128), jnp.int32),
            pltpu.VMEM((CHM, 128), jnp.int32),
            pltpu.VMEM((128, D), jnp.float32),
            pltpu.VMEM((128, D), jnp.float32),
            pltpu.VMEM_SHARED((NPAD, D), jnp.float32),
            pltpu.VMEM_SHARED((NPAD, D), jnp.float32),
            pltpu.SemaphoreType.DMA,
            pltpu.SemaphoreType.DMA,
            pltpu.SemaphoreType.DMA,
            pltpu.SemaphoreType.DMA,
        ],
        compiler_params=pltpu.CompilerParams(needs_layout_passes=False,
                                             use_tc_tiling_on_sc=False),
    )
    def _scatter(srcw, dstw, g, out, src_l, dst_l, buf0, buf1, accum, gsh,
                 gsem0, gsem1, ssem0, ssem1):
        c = lax.axis_index("c")
        s = lax.axis_index("s")
        w = c * NS + s
        cbase, extra = _worker_chunks(w)
        _load_idx(srcw, src_l, cbase, w)
        _load_idx(dstw, dst_l, cbase, w)

        stripe = pl.ds(s * STR, STR)
        pltpu.async_copy(g.at[stripe], gsh.at[stripe], gsem1)

        zeros = jnp.zeros((16,), jnp.float32)

        def _zb(i, _):
            for k in range(D // 16):
                buf0[i, pl.ds(k * 16, 16)] = zeros
            return 0

        lax.fori_loop(0, 128, _zb, 0)
        for k in range(STR // 128):
            pltpu.sync_copy(buf0, accum.at[pl.ds(s * STR + k * 128, 128)])
        pltpu.make_async_copy(g.at[stripe], gsh.at[stripe], gsem1).wait()
        plsc.subcore_barrier()

        pltpu.async_copy(gsh.at[src_l.at[0]], buf0, gsem0)
        T = CHB // 2

        def _edge_pair(t, _):
            j0 = 2 * t
            j1 = 2 * t + 1
            pltpu.make_async_copy(gsh.at[src_l.at[j0]], buf0, gsem0).wait()

            @pl.when(t > 0)
            def _():
                pltpu.make_async_copy(buf1, accum.at[dst_l.at[j1]],
                                      ssem1).wait()

            pltpu.async_copy(gsh.at[src_l.at[j1]], buf1, gsem1)
            pltpu.async_copy(buf0, accum.at[dst_l.at[j0]], ssem0, add=True)
            pltpu.make_async_copy(gsh.at[src_l.at[j1]], buf1, gsem1).wait()
            pltpu.make_async_copy(buf0, accum.at[dst_l.at[j0]], ssem0).wait()

            @pl.when(t < T - 1)
            def _():
                pltpu.async_copy(gsh.at[src_l.at[j0 + 2]], buf0, gsem0)

            pltpu.async_copy(buf1, accum.at[dst_l.at[j1]], ssem1, add=True)
            return 0

        lax.fori_loop(0, T, _edge_pair, 0)
        pltpu.make_async_copy(buf1, accum.at[dst_l.at[CHB - 1]], ssem1).wait()

        @pl.when(extra > 0)
        def _():
            pltpu.async_copy(gsh.at[src_l.at[CHB]], buf0, gsem0).wait()
            pltpu.sync_copy(buf0, accum.at[dst_l.at[CHB]], add=True)

        plsc.subcore_barrier()
        pltpu.sync_copy(
            accum.at[pl.ds(s * STR, STR)], out.at[c, pl.ds(s * STR, STR)]
        )

    return _scatter


def _dinv_of(degp_blk):
    deg = degp_blk[0, :] + degp_blk[1, :] + 1.0
    return lax.rsqrt(jnp.maximum(deg, 1.0))


def _enc1_body(x_ref, w1_ref, degp_ref, out_ref):
    dinv = _dinv_of(degp_ref)
    h = jnp.dot(x_ref[...], w1_ref[...], preferred_element_type=jnp.float32,
                precision=lax.Precision.HIGHEST)
    out_ref[...] = h * dinv[:, None]


def _enc1(x, W1, degp):
    return pl.pallas_call(
        _enc1_body,
        grid=(NPAD // RB,),
        in_specs=[
            pl.BlockSpec((RB, 128), lambda i: (i, 0)),
            pl.BlockSpec((128, 64), lambda i: (0, 0)),
            pl.BlockSpec((NC, RB), lambda i: (0, i)),
        ],
        out_specs=pl.BlockSpec((RB, 64), lambda i: (i, 0)),
        out_shape=jax.ShapeDtypeStruct((NPAD, 64), jnp.float32),
    )(x, W1, degp)


def _enc2_body(s1_ref, g1_ref, degp_ref, b1_ref, w2_ref, out_ref):
    dinv = _dinv_of(degp_ref)
    conv = dinv[:, None] * (s1_ref[0] + s1_ref[1] + g1_ref[...]) + b1_ref[...]
    h = jnp.maximum(conv, 0.0)
    t = jnp.dot(h, w2_ref[...], preferred_element_type=jnp.float32,
                precision=lax.Precision.HIGHEST)
    out_ref[...] = t * dinv[:, None]


def _enc2(S1, g1, degp, b1, W2):
    return pl.pallas_call(
        _enc2_body,
        grid=(NPAD // RB,),
        in_specs=[
            pl.BlockSpec((NC, RB, 64), lambda i: (0, i, 0)),
            pl.BlockSpec((RB, 64), lambda i: (i, 0)),
            pl.BlockSpec((NC, RB), lambda i: (0, i)),
            pl.BlockSpec((1, 64), lambda i: (0, 0)),
            pl.BlockSpec((64, 32), lambda i: (0, 0)),
        ],
        out_specs=pl.BlockSpec((RB, 32), lambda i: (i, 0)),
        out_shape=jax.ShapeDtypeStruct((NPAD, 32), jnp.float32),
    )(S1, g1, degp, b1, W2)


def _dec_body(s2_ref, g2_ref, degp_ref, b2_ref, wd1_ref, bd1_ref, wd2_ref,
              bd2_ref, eps_ref, dec_ref, mu_ref, lv_ref):
    dinv = _dinv_of(degp_ref)
    enc = dinv[:, None] * (s2_ref[0] + s2_ref[1] + g2_ref[...]) + b2_ref[...]
    mu = enc[:, :16]
    lv = enc[:, 16:]
    mu_ref[...] = mu
    lv_ref[...] = lv
    std = jnp.exp(0.5 * lv)
    z = mu + eps_ref[...] * std
    d = jnp.dot(z, wd1_ref[...], preferred_element_type=jnp.float32,
                precision=lax.Precision.HIGHEST) + bd1_ref[...]
    d = jnp.maximum(d, 0.0)
    o = jnp.dot(d, wd2_ref[...], preferred_element_type=jnp.float32,
                precision=lax.Precision.HIGHEST) + bd2_ref[...]
    dec_ref[...] = jax.nn.sigmoid(o)


def _dec(S2, g2, degp, b2, Wd1, bd1, Wd2, bd2, eps):
    return pl.pallas_call(
        _dec_body,
        grid=(NPAD // RB,),
        in_specs=[
            pl.BlockSpec((NC, RB, 32), lambda i: (0, i, 0)),
            pl.BlockSpec((RB, 32), lambda i: (i, 0)),
            pl.BlockSpec((NC, RB), lambda i: (0, i)),
            pl.BlockSpec((1, 32), lambda i: (0, 0)),
            pl.BlockSpec((16, 64), lambda i: (0, 0)),
            pl.BlockSpec((1, 64), lambda i: (0, 0)),
            pl.BlockSpec((64, 128), lambda i: (0, 0)),
            pl.BlockSpec((1, 128), lambda i: (0, 0)),
            pl.BlockSpec((RB, 16), lambda i: (i, 0)),
        ],
        out_specs=[
            pl.BlockSpec((RB, 128), lambda i: (i, 0)),
            pl.BlockSpec((RB, 16), lambda i: (i, 0)),
            pl.BlockSpec((RB, 16), lambda i: (i, 0)),
        ],
        out_shape=[
            jax.ShapeDtypeStruct((N, 128), jnp.float32),
            jax.ShapeDtypeStruct((N, 16), jnp.float32),
            jax.ShapeDtypeStruct((N, 16), jnp.float32),
        ],
    )(S2, g2, degp, b2, Wd1, bd1, Wd2, bd2, eps)


@jax.jit
def kernel(x, edge_index, W1, b1, W2, b2, Wd1, bd1, Wd2, bd2):
    ei = edge_index.astype(jnp.int32)
    srcw = ei[0].reshape(NCH, 128)
    dstw = ei[1].reshape(NCH, 128)
    srcw, dstw = lax.optimization_barrier((srcw, dstw))
    eps = jnp.asarray(_EPS)

    degp = _make_deg()(dstw)
    g1 = _enc1(x, W1, degp)
    S1 = _make_scatter(64)(srcw, dstw, g1)
    g2 = _enc2(S1, g1, degp, b1.reshape(1, 64), W2)
    S2 = _make_scatter(32)(srcw, dstw, g2)
    dec, mu, lv = _dec(S2, g2, degp, b2.reshape(1, 32), Wd1,
                       bd1.reshape(1, 64), Wd2, bd2.reshape(1, 128), eps)
    return (dec, mu, lv)


# speedup vs baseline: 1.1000x; 1.0453x over previous
import functools

import jax
import jax.numpy as jnp
import numpy as np
from jax import lax
from jax.experimental import pallas as pl
from jax.experimental.pallas import tpu as pltpu
from jax.experimental.pallas import tpu_sc as plsc

N = 10000
E = 320000
NPAD = 10240
STR = 640
NC = 2
NS = 16
NWK = NC * NS
NCH = E // 128
CHB = NCH // NWK
XW = NCH - CHB * NWK
CHM = CHB + 1
RB = 2048

_EPS = np.asarray(
    jax.random.normal(jax.random.key(42), (N, 16), dtype=jnp.float32))


@functools.cache
def _mesh():
    return plsc.VectorSubcoreMesh(core_axis_name="c", subcore_axis_name="s",
                                  num_cores=NC, num_subcores=NS)


def _worker_chunks(w):
    cbase = CHB * w + jnp.minimum(w, XW)
    extra = (w < XW).astype(jnp.int32)
    return cbase, extra


def _load_idx(hbm_flat, vmem, cbase, w, off):
    base = off + cbase * 128

    @pl.when(w < XW)
    def _():
        pltpu.sync_copy(hbm_flat.at[pl.ds(base, CHM * 128)], vmem)

    @pl.when(w >= XW)
    def _():
        pltpu.sync_copy(hbm_flat.at[pl.ds(base, CHB * 128)],
                        vmem.at[pl.ds(0, CHB * 128)])


def _chunk(vmem_flat, j):
    return vmem_flat.at[pl.ds(j * 128, 128)]


@functools.cache
def _make_deg():
    return functools.partial(
        pl.kernel,
        out_type=jax.ShapeDtypeStruct((NC, NPAD), jnp.float32),
        mesh=_mesh(),
        scratch_types=[
            pltpu.VMEM((CHM * 128,), jnp.int32),
            pltpu.VMEM((NPAD,), jnp.float32),
            pltpu.VMEM((STR,), jnp.float32),
            pltpu.VMEM((STR,), jnp.float32),
            pltpu.VMEM_SHARED((NS, NPAD), jnp.float32),
        ],
        compiler_params=pltpu.CompilerParams(needs_layout_passes=False,
                                             use_tc_tiling_on_sc=False),
    )(_deg_body)


def _deg_body(eflat, outd, ids, degl, acc, tmp, degsh):
    c = lax.axis_index("c")
    s = lax.axis_index("s")
    w = c * NS + s
    cbase, extra = _worker_chunks(w)
    _load_idx(eflat, ids, cbase, w, E)
    zeros = jnp.zeros((16,), jnp.float32)
    ones = jnp.ones((16,), jnp.float32)

    def _zero(i, _):
        degl[pl.ds(i * 16, 16)] = zeros
        return 0

    lax.fori_loop(0, NPAD // 16, _zero, 0)

    def _count(r, _):
        for k in range(8):
            idx = ids[pl.ds(r * 128 + k * 16, 16)]
            plsc.addupdate_scatter(degl, [idx], ones)
        return 0

    lax.fori_loop(0, CHB + extra, _count, 0)
    pltpu.sync_copy(degl, degsh.at[s])
    plsc.subcore_barrier()

    def _zacc(i, _):
        acc[pl.ds(i * 16, 16)] = zeros
        return 0

    lax.fori_loop(0, STR // 16, _zacc, 0)

    def _red(t, _):
        pltpu.sync_copy(degsh.at[t, pl.ds(s * STR, STR)], tmp)

        def _add(q, _):
            sl = pl.ds(q * 16, 16)
            acc[sl] = acc[sl] + tmp[sl]
            return 0

        lax.fori_loop(0, STR // 16, _add, 0)
        return 0

    lax.fori_loop(0, NS, _red, 0)
    pltpu.sync_copy(acc, outd.at[c, pl.ds(s * STR, STR)])


@functools.cache
def _make_scatter(D):
    @functools.partial(
        pl.kernel,
        out_type=jax.ShapeDtypeStruct((NC, NPAD, D), jnp.float32),
        mesh=_mesh(),
        scratch_types=[
            pltpu.VMEM((CHM * 128,), jnp.int32),
            pltpu.VMEM((CHM * 128,), jnp.int32),
            pltpu.VMEM((128, D), jnp.float32),
            pltpu.VMEM((128, D), jnp.float32),
            pltpu.VMEM_SHARED((NPAD, D), jnp.float32),
            pltpu.VMEM_SHARED((NPAD, D), jnp.float32),
            pltpu.SemaphoreType.DMA,
            pltpu.SemaphoreType.DMA,
            pltpu.SemaphoreType.DMA,
            pltpu.SemaphoreType.DMA,
        ],
        compiler_params=pltpu.CompilerParams(needs_layout_passes=False,
                                             use_tc_tiling_on_sc=False),
    )
    def _scatter(eflat, g, out, src_l, dst_l, buf0, buf1, accum, gsh,
                 gsem0, gsem1, ssem0, ssem1):
        c = lax.axis_index("c")
        s = lax.axis_index("s")
        w = c * NS + s
        cbase, extra = _worker_chunks(w)
        _load_idx(eflat, src_l, cbase, w, 0)
        _load_idx(eflat, dst_l, cbase, w, E)

        stripe = pl.ds(s * STR, STR)
        pltpu.async_copy(g.at[stripe], gsh.at[stripe], gsem1)

        zeros = jnp.zeros((16,), jnp.float32)

        def _zb(i, _):
            for k in range(D // 16):
                buf0[i, pl.ds(k * 16, 16)] = zeros
            return 0

        lax.fori_loop(0, 128, _zb, 0)
        for k in range(STR // 128):
            pltpu.sync_copy(buf0, accum.at[pl.ds(s * STR + k * 128, 128)])
        pltpu.make_async_copy(g.at[stripe], gsh.at[stripe], gsem1).wait()
        plsc.subcore_barrier()

        pltpu.async_copy(gsh.at[_chunk(src_l, 0)], buf0, gsem0)
        T = CHB // 2

        def _edge_pair(t, _):
            j0 = 2 * t
            j1 = 2 * t + 1
            pltpu.make_async_copy(gsh.at[_chunk(src_l, j0)], buf0,
                                  gsem0).wait()

            @pl.when(t > 0)
            def _():
                pltpu.make_async_copy(buf1, accum.at[_chunk(dst_l, j1)],
                                      ssem1).wait()

            pltpu.async_copy(gsh.at[_chunk(src_l, j1)], buf1, gsem1)
            pltpu.async_copy(buf0, accum.at[_chunk(dst_l, j0)], ssem0,
                             add=True)
            pltpu.make_async_copy(gsh.at[_chunk(src_l, j1)], buf1,
                                  gsem1).wait()
            pltpu.make_async_copy(buf0, accum.at[_chunk(dst_l, j0)],
                                  ssem0).wait()

            @pl.when(t < T - 1)
            def _():
                pltpu.async_copy(gsh.at[_chunk(src_l, j0 + 2)], buf0, gsem0)

            pltpu.async_copy(buf1, accum.at[_chunk(dst_l, j1)], ssem1,
                             add=True)
            return 0

        lax.fori_loop(0, T, _edge_pair, 0)
        pltpu.make_async_copy(buf1, accum.at[_chunk(dst_l, CHB - 1)],
                              ssem1).wait()

        @pl.when(extra > 0)
        def _():
            pltpu.async_copy(gsh.at[_chunk(src_l, CHB)], buf0, gsem0).wait()
            pltpu.sync_copy(buf0, accum.at[_chunk(dst_l, CHB)], add=True)

        plsc.subcore_barrier()
        pltpu.sync_copy(
            accum.at[pl.ds(s * STR, STR)], out.at[c, pl.ds(s * STR, STR)]
        )

    return _scatter


def _dinv_of(degp_blk):
    deg = degp_blk[0, :] + degp_blk[1, :] + 1.0
    return lax.rsqrt(jnp.maximum(deg, 1.0))


def _enc1_body(x_ref, w1_ref, degp_ref, out_ref):
    dinv = _dinv_of(degp_ref)
    h = jnp.dot(x_ref[...], w1_ref[...], preferred_element_type=jnp.float32,
                precision=lax.Precision.HIGHEST)
    out_ref[...] = h * dinv[:, None]


def _enc1(x, W1, degp):
    return pl.pallas_call(
        _enc1_body,
        grid=(NPAD // RB,),
        in_specs=[
            pl.BlockSpec((RB, 128), lambda i: (i, 0)),
            pl.BlockSpec((128, 64), lambda i: (0, 0)),
            pl.BlockSpec((NC, RB), lambda i: (0, i)),
        ],
        out_specs=pl.BlockSpec((RB, 64), lambda i: (i, 0)),
        out_shape=jax.ShapeDtypeStruct((NPAD, 64), jnp.float32),
    )(x, W1, degp)


def _enc2_body(s1_ref, g1_ref, degp_ref, b1_ref, w2_ref, out_ref):
    dinv = _dinv_of(degp_ref)
    conv = dinv[:, None] * (s1_ref[0] + s1_ref[1] + g1_ref[...]) + b1_ref[...]
    h = jnp.maximum(conv, 0.0)
    t = jnp.dot(h, w2_ref[...], preferred_element_type=jnp.float32,
                precision=lax.Precision.HIGHEST)
    out_ref[...] = t * dinv[:, None]


def _enc2(S1, g1, degp, b1, W2):
    return pl.pallas_call(
        _enc2_body,
        grid=(NPAD // RB,),
        in_specs=[
            pl.BlockSpec((NC, RB, 64), lambda i: (0, i, 0)),
            pl.BlockSpec((RB, 64), lambda i: (i, 0)),
            pl.BlockSpec((NC, RB), lambda i: (0, i)),
            pl.BlockSpec((1, 64), lambda i: (0, 0)),
            pl.BlockSpec((64, 32), lambda i: (0, 0)),
        ],
        out_specs=pl.BlockSpec((RB, 32), lambda i: (i, 0)),
        out_shape=jax.ShapeDtypeStruct((NPAD, 32), jnp.float32),
    )(S1, g1, degp, b1, W2)


def _dec_body(s2_ref, g2_ref, degp_ref, b2_ref, wd1_ref, bd1_ref, wd2_ref,
              bd2_ref, eps_ref, dec_ref, mu_ref, lv_ref):
    dinv = _dinv_of(degp_ref)
    enc = dinv[:, None] * (s2_ref[0] + s2_ref[1] + g2_ref[...]) + b2_ref[...]
    mu = enc[:, :16]
    lv = enc[:, 16:]
    mu_ref[...] = mu
    lv_ref[...] = lv
    std = jnp.exp(0.5 * lv)
    z = mu + eps_ref[...] * std
    d = jnp.dot(z, wd1_ref[...], preferred_element_type=jnp.float32,
                precision=lax.Precision.HIGHEST) + bd1_ref[...]
    d = jnp.maximum(d, 0.0)
    o = jnp.dot(d, wd2_ref[...], preferred_element_type=jnp.float32,
                precision=lax.Precision.HIGHEST) + bd2_ref[...]
    dec_ref[...] = jax.nn.sigmoid(o)


def _dec(S2, g2, degp, b2, Wd1, bd1, Wd2, bd2, eps):
    return pl.pallas_call(
        _dec_body,
        grid=(NPAD // RB,),
        in_specs=[
            pl.BlockSpec((NC, RB, 32), lambda i: (0, i, 0)),
            pl.BlockSpec((RB, 32), lambda i: (i, 0)),
            pl.BlockSpec((NC, RB), lambda i: (0, i)),
            pl.BlockSpec((1, 32), lambda i: (0, 0)),
            pl.BlockSpec((16, 64), lambda i: (0, 0)),
            pl.BlockSpec((1, 64), lambda i: (0, 0)),
            pl.BlockSpec((64, 128), lambda i: (0, 0)),
            pl.BlockSpec((1, 128), lambda i: (0, 0)),
            pl.BlockSpec((RB, 16), lambda i: (i, 0)),
        ],
        out_specs=[
            pl.BlockSpec((RB, 128), lambda i: (i, 0)),
            pl.BlockSpec((RB, 16), lambda i: (i, 0)),
            pl.BlockSpec((RB, 16), lambda i: (i, 0)),
        ],
        out_shape=[
            jax.ShapeDtypeStruct((N, 128), jnp.float32),
            jax.ShapeDtypeStruct((N, 16), jnp.float32),
            jax.ShapeDtypeStruct((N, 16), jnp.float32),
        ],
    )(S2, g2, degp, b2, Wd1, bd1, Wd2, bd2, eps)


@jax.jit
def kernel(x, edge_index, W1, b1, W2, b2, Wd1, bd1, Wd2, bd2):
    eflat = lax.optimization_barrier(edge_index.astype(jnp.int32).reshape(-1))
    eps = jnp.asarray(_EPS)

    degp = _make_deg()(eflat)
    g1 = _enc1(x, W1, degp)
    S1 = _make_scatter(64)(eflat, g1)
    g2 = _enc2(S1, g1, degp, b1.reshape(1, 64), W2)
    S2 = _make_scatter(32)(eflat, g2)
    dec, mu, lv = _dec(S2, g2, degp, b2.reshape(1, 32), Wd1,
                       bd1.reshape(1, 64), Wd2, bd2.reshape(1, 128), eps)
    return (dec, mu, lv)


# speedup vs baseline: 1.2099x; 1.0999x over previous
import functools

import jax
import jax.numpy as jnp
import numpy as np
from jax import lax
from jax.experimental import pallas as pl
from jax.experimental.pallas import tpu as pltpu
from jax.experimental.pallas import tpu_sc as plsc

N = 10000
E = 320000
NPAD = 10240
STR = 640
NC = 2
NS = 16
NWK = NC * NS
NCH = E // 128
CHB = NCH // NWK
XW = NCH - CHB * NWK
CHM = CHB + 1
RB = 2048

_EPS = np.asarray(
    jax.random.normal(jax.random.key(42), (N, 16), dtype=jnp.float32))


@functools.cache
def _mesh():
    return plsc.VectorSubcoreMesh(core_axis_name="c", subcore_axis_name="s",
                                  num_cores=NC, num_subcores=NS)


def _worker_chunks(w):
    cbase = CHB * w + jnp.minimum(w, XW)
    extra = (w < XW).astype(jnp.int32)
    return cbase, extra


def _load_idx(hbm_flat, vmem, cbase, w, off):
    base = off + cbase * 128

    @pl.when(w < XW)
    def _():
        pltpu.sync_copy(hbm_flat.at[pl.ds(base, CHM * 128)], vmem)

    @pl.when(w >= XW)
    def _():
        pltpu.sync_copy(hbm_flat.at[pl.ds(base, CHB * 128)],
                        vmem.at[pl.ds(0, CHB * 128)])


def _chunk(vmem_flat, j):
    return vmem_flat.at[pl.ds(j * 128, 128)]


@functools.cache
def _make_deg():
    return functools.partial(
        pl.kernel,
        out_type=jax.ShapeDtypeStruct((NC, NPAD), jnp.float32),
        mesh=_mesh(),
        scratch_types=[
            pltpu.VMEM((CHM * 128,), jnp.int32),
            pltpu.VMEM((NPAD,), jnp.float32),
            pltpu.VMEM((STR,), jnp.float32),
            pltpu.VMEM((STR,), jnp.float32),
            pltpu.VMEM_SHARED((NS, NPAD), jnp.float32),
        ],
        compiler_params=pltpu.CompilerParams(needs_layout_passes=False,
                                             use_tc_tiling_on_sc=False),
    )(_deg_body)


def _deg_body(eflat, outd, ids, degl, acc, tmp, degsh):
    c = lax.axis_index("c")
    s = lax.axis_index("s")
    w = c * NS + s
    cbase, extra = _worker_chunks(w)
    _load_idx(eflat, ids, cbase, w, E)
    zeros = jnp.zeros((16,), jnp.float32)
    ones = jnp.ones((16,), jnp.float32)

    def _zero(i, _):
        degl[pl.ds(i * 16, 16)] = zeros
        return 0

    lax.fori_loop(0, NPAD // 16, _zero, 0)

    def _count(r, _):
        for k in range(8):
            idx = ids[pl.ds(r * 128 + k * 16, 16)]
            plsc.addupdate_scatter(degl, [idx], ones)
        return 0

    lax.fori_loop(0, CHB + extra, _count, 0)
    pltpu.sync_copy(degl, degsh.at[s])
    plsc.subcore_barrier()

    def _zacc(i, _):
        acc[pl.ds(i * 16, 16)] = zeros
        return 0

    lax.fori_loop(0, STR // 16, _zacc, 0)

    def _red(t, _):
        pltpu.sync_copy(degsh.at[t, pl.ds(s * STR, STR)], tmp)

        def _add(q, _):
            sl = pl.ds(q * 16, 16)
            acc[sl] = acc[sl] + tmp[sl]
            return 0

        lax.fori_loop(0, STR // 16, _add, 0)
        return 0

    lax.fori_loop(0, NS, _red, 0)
    pltpu.sync_copy(acc, outd.at[c, pl.ds(s * STR, STR)])


@functools.cache
def _make_scatter(D):
    @functools.partial(
        pl.kernel,
        out_type=jax.ShapeDtypeStruct((NC, NPAD, 128), jnp.float32),
        mesh=_mesh(),
        scratch_types=[
            pltpu.VMEM((CHM * 128,), jnp.int32),
            pltpu.VMEM((CHM * 128,), jnp.int32),
            pltpu.VMEM((128, D), jnp.float32),
            pltpu.VMEM((128, D), jnp.float32),
            pltpu.VMEM_SHARED((NPAD, D), jnp.float32),
            pltpu.VMEM_SHARED((NPAD, D), jnp.float32),
            pltpu.SemaphoreType.DMA,
            pltpu.SemaphoreType.DMA,
            pltpu.SemaphoreType.DMA,
            pltpu.SemaphoreType.DMA,
        ],
        compiler_params=pltpu.CompilerParams(needs_layout_passes=False,
                                             use_tc_tiling_on_sc=False),
    )
    def _scatter(eflat, g, out, src_l, dst_l, buf0, buf1, accum, gsh,
                 gsem0, gsem1, ssem0, ssem1):
        c = lax.axis_index("c")
        s = lax.axis_index("s")
        w = c * NS + s
        cbase, extra = _worker_chunks(w)
        _load_idx(eflat, src_l, cbase, w, 0)
        _load_idx(eflat, dst_l, cbase, w, E)

        stripe = pl.ds(s * STR, STR)
        pltpu.async_copy(g.at[stripe, pl.ds(0, D)], gsh.at[stripe], gsem1)

        zeros = jnp.zeros((16,), jnp.float32)

        def _zb(i, _):
            for k in range(D // 16):
                buf0[i, pl.ds(k * 16, 16)] = zeros
            return 0

        lax.fori_loop(0, 128, _zb, 0)
        for k in range(STR // 128):
            pltpu.sync_copy(buf0, accum.at[pl.ds(s * STR + k * 128, 128)])
        pltpu.make_async_copy(g.at[stripe, pl.ds(0, D)], gsh.at[stripe],
                              gsem1).wait()
        plsc.subcore_barrier()

        pltpu.async_copy(gsh.at[_chunk(src_l, 0)], buf0, gsem0)
        T = CHB // 2

        def _edge_pair(t, _):
            j0 = 2 * t
            j1 = 2 * t + 1
            pltpu.make_async_copy(gsh.at[_chunk(src_l, j0)], buf0,
                                  gsem0).wait()

            @pl.when(t > 0)
            def _():
                pltpu.make_async_copy(buf1, accum.at[_chunk(dst_l, j1)],
                                      ssem1).wait()

            pltpu.async_copy(gsh.at[_chunk(src_l, j1)], buf1, gsem1)
            pltpu.async_copy(buf0, accum.at[_chunk(dst_l, j0)], ssem0,
                             add=True)
            pltpu.make_async_copy(gsh.at[_chunk(src_l, j1)], buf1,
                                  gsem1).wait()
            pltpu.make_async_copy(buf0, accum.at[_chunk(dst_l, j0)],
                                  ssem0).wait()

            @pl.when(t < T - 1)
            def _():
                pltpu.async_copy(gsh.at[_chunk(src_l, j0 + 2)], buf0, gsem0)

            pltpu.async_copy(buf1, accum.at[_chunk(dst_l, j1)], ssem1,
                             add=True)
            return 0

        lax.fori_loop(0, T, _edge_pair, 0)
        pltpu.make_async_copy(buf1, accum.at[_chunk(dst_l, CHB - 1)],
                              ssem1).wait()

        @pl.when(extra > 0)
        def _():
            pltpu.async_copy(gsh.at[_chunk(src_l, CHB)], buf0, gsem0).wait()
            pltpu.sync_copy(buf0, accum.at[_chunk(dst_l, CHB)], add=True)

        plsc.subcore_barrier()
        pltpu.sync_copy(
            accum.at[pl.ds(s * STR, STR)],
            out.at[c, pl.ds(s * STR, STR), pl.ds(0, D)],
        )

    return _scatter


def _dinv_of(degp_blk):
    deg = degp_blk[0, :] + degp_blk[1, :] + 1.0
    return lax.rsqrt(jnp.maximum(deg, 1.0))


def _enc1_body(x_ref, w1_ref, degp_ref, out_ref):
    dinv = _dinv_of(degp_ref)
    h = jnp.dot(x_ref[...], w1_ref[...], preferred_element_type=jnp.float32,
                precision=lax.Precision.HIGHEST)
    out_ref[:, :64] = h * dinv[:, None]


def _enc1(x, W1, degp):
    return pl.pallas_call(
        _enc1_body,
        grid=(NPAD // RB,),
        in_specs=[
            pl.BlockSpec((RB, 128), lambda i: (i, 0)),
            pl.BlockSpec((128, 64), lambda i: (0, 0)),
            pl.BlockSpec((NC, RB), lambda i: (0, i)),
        ],
        out_specs=pl.BlockSpec((RB, 128), lambda i: (i, 0)),
        out_shape=jax.ShapeDtypeStruct((NPAD, 128), jnp.float32),
    )(x, W1, degp)


def _enc2_body(s1_ref, g1_ref, degp_ref, b1_ref, w2_ref, out_ref):
    dinv = _dinv_of(degp_ref)
    conv = (dinv[:, None] * (s1_ref[0, :, :64] + s1_ref[1, :, :64]
                             + g1_ref[:, :64]) + b1_ref[...])
    h = jnp.maximum(conv, 0.0)
    t = jnp.dot(h, w2_ref[...], preferred_element_type=jnp.float32,
                precision=lax.Precision.HIGHEST)
    out_ref[:, :32] = t * dinv[:, None]


def _enc2(S1, g1, degp, b1, W2):
    return pl.pallas_call(
        _enc2_body,
        grid=(NPAD // RB,),
        in_specs=[
            pl.BlockSpec((NC, RB, 128), lambda i: (0, i, 0)),
            pl.BlockSpec((RB, 128), lambda i: (i, 0)),
            pl.BlockSpec((NC, RB), lambda i: (0, i)),
            pl.BlockSpec((1, 64), lambda i: (0, 0)),
            pl.BlockSpec((64, 32), lambda i: (0, 0)),
        ],
        out_specs=pl.BlockSpec((RB, 128), lambda i: (i, 0)),
        out_shape=jax.ShapeDtypeStruct((NPAD, 128), jnp.float32),
    )(S1, g1, degp, b1, W2)


def _dec_body(s2_ref, g2_ref, degp_ref, b2_ref, wd1_ref, bd1_ref, wd2_ref,
              bd2_ref, eps_ref, dec_ref, mu_ref, lv_ref):
    dinv = _dinv_of(degp_ref)
    enc = (dinv[:, None] * (s2_ref[0, :, :32] + s2_ref[1, :, :32]
                            + g2_ref[:, :32]) + b2_ref[...])
    mu = enc[:, :16]
    lv = enc[:, 16:]
    mu_ref[...] = mu
    lv_ref[...] = lv
    std = jnp.exp(0.5 * lv)
    z = mu + eps_ref[...] * std
    d = jnp.dot(z, wd1_ref[...], preferred_element_type=jnp.float32,
                precision=lax.Precision.HIGHEST) + bd1_ref[...]
    d = jnp.maximum(d, 0.0)
    o = jnp.dot(d, wd2_ref[...], preferred_element_type=jnp.float32,
                precision=lax.Precision.HIGHEST) + bd2_ref[...]
    dec_ref[...] = jax.nn.sigmoid(o)


def _dec(S2, g2, degp, b2, Wd1, bd1, Wd2, bd2, eps):
    return pl.pallas_call(
        _dec_body,
        grid=(NPAD // RB,),
        in_specs=[
            pl.BlockSpec((NC, RB, 128), lambda i: (0, i, 0)),
            pl.BlockSpec((RB, 128), lambda i: (i, 0)),
            pl.BlockSpec((NC, RB), lambda i: (0, i)),
            pl.BlockSpec((1, 32), lambda i: (0, 0)),
            pl.BlockSpec((16, 64), lambda i: (0, 0)),
            pl.BlockSpec((1, 64), lambda i: (0, 0)),
            pl.BlockSpec((64, 128), lambda i: (0, 0)),
            pl.BlockSpec((1, 128), lambda i: (0, 0)),
            pl.BlockSpec((RB, 16), lambda i: (i, 0)),
        ],
        out_specs=[
            pl.BlockSpec((RB, 128), lambda i: (i, 0)),
            pl.BlockSpec((RB, 16), lambda i: (i, 0)),
            pl.BlockSpec((RB, 16), lambda i: (i, 0)),
        ],
        out_shape=[
            jax.ShapeDtypeStruct((N, 128), jnp.float32),
            jax.ShapeDtypeStruct((N, 16), jnp.float32),
            jax.ShapeDtypeStruct((N, 16), jnp.float32),
        ],
    )(S2, g2, degp, b2, Wd1, bd1, Wd2, bd2, eps)


@jax.jit
def kernel(x, edge_index, W1, b1, W2, b2, Wd1, bd1, Wd2, bd2):
    eflat = lax.optimization_barrier(edge_index.astype(jnp.int32).reshape(-1))
    eps = jnp.asarray(_EPS)

    degp = _make_deg()(eflat)
    g1 = _enc1(x, W1, degp)
    S1 = _make_scatter(64)(eflat, g1)
    g2 = _enc2(S1, g1, degp, b1.reshape(1, 64), W2)
    S2 = _make_scatter(32)(eflat, g2)
    dec, mu, lv = _dec(S2, g2, degp, b2.reshape(1, 32), Wd1,
                       bd1.reshape(1, 64), Wd2, bd2.reshape(1, 128), eps)
    return (dec, mu, lv)
